# Initial kernel scaffold; baseline (speedup 1.0000x reference)
import jax
import jax.numpy as jnp
from jax.experimental import pallas as pl


def kernel(x, embedding, W_ih0, W_hh0, b0, W_ih1, W_hh1, b1):
    raise NotImplementedError("write your pallas kernel here")


# speedup vs baseline: 3.5908x; 3.5908x over previous
import functools

import jax
import jax.numpy as jnp
from jax import lax
from jax.experimental import pallas as pl
from jax.experimental.pallas import tpu as pltpu
from jax.experimental.pallas import tpu_sc as plsc

B = 16
S = 512
EMB = 512
HID = 1024

T_CHUNK = 64
N_CHUNKS = S // T_CHUNK

SC_NC = 2
SC_NS = 16
SC_NW = SC_NC * SC_NS
GATHER_ROWS = B * S
ROWS_PER_W = GATHER_ROWS // SC_NW
GCHUNK = 64


def _sc_gather(table, idx):
    n, d = idx.shape[0], table.shape[1]
    mesh = plsc.VectorSubcoreMesh(core_axis_name="c", subcore_axis_name="s")

    @functools.partial(
        pl.kernel,
        mesh=mesh,
        out_type=jax.ShapeDtypeStruct((n, d), table.dtype),
        scratch_types=[
            pltpu.VMEM((GCHUNK,), jnp.int32),
            pltpu.VMEM((GCHUNK, d), table.dtype),
            pltpu.SemaphoreType.DMA,
        ],
    )
    def gather_kernel(table_hbm, idx_hbm, out_hbm, idx_v, rows_v, sem):
        wid = lax.axis_index("s") * SC_NC + lax.axis_index("c")
        base = wid * ROWS_PER_W

        @pl.loop(0, ROWS_PER_W // GCHUNK)
        def _(j):
            off = base + j * GCHUNK
            pltpu.sync_copy(idx_hbm.at[pl.ds(off, GCHUNK)], idx_v)
            pltpu.async_copy(table_hbm.at[idx_v], rows_v, sem).wait()
            pltpu.sync_copy(rows_v, out_hbm.at[pl.ds(off, GCHUNK)])

    return gather_kernel(table, idx)


def _matmul_kernel(x_ref, w_ref, o_ref):
    o_ref[...] = jnp.dot(x_ref[...], w_ref[...],
                         preferred_element_type=jnp.float32)


def _matmul(x, w, block_m=1024):
    m, k = x.shape
    n = w.shape[1]
    return pl.pallas_call(
        _matmul_kernel,
        grid=(m // block_m,),
        in_specs=[
            pl.BlockSpec((block_m, k), lambda g: (g, 0)),
            pl.BlockSpec((k, n), lambda g: (0, 0)),
        ],
        out_specs=pl.BlockSpec((block_m, n), lambda g: (g, 0)),
        out_shape=jax.ShapeDtypeStruct((m, n), jnp.float32),
    )(x, w)


def _recurrence_kernel(a_ref, m_ref, w_ref, b_ref, hall_ref, hfin_ref,
                       h_scratch):
    g = pl.program_id(0)

    @pl.when(g == 0)
    def _():
        h_scratch[...] = jnp.zeros_like(h_scratch)

    w = w_ref[...]
    b = b_ref[...]

    def step(t, h):
        a = a_ref[pl.ds(t * B, B), :]
        h_new = jnp.tanh(a + jnp.dot(h, w, preferred_element_type=jnp.float32)
                         + b)
        hall_ref[pl.ds(t * B, B), :] = h_new
        m = m_ref[pl.ds(t * B, B), :]
        return jnp.where(m > 0, h_new, h)

    h = lax.fori_loop(0, T_CHUNK, step, h_scratch[...])
    h_scratch[...] = h

    @pl.when(g == pl.num_programs(0) - 1)
    def _():
        hfin_ref[...] = h


def _recurrence(a, mask, w_hh, b):
    return pl.pallas_call(
        _recurrence_kernel,
        grid=(N_CHUNKS,),
        in_specs=[
            pl.BlockSpec((T_CHUNK * B, HID), lambda g: (g, 0)),
            pl.BlockSpec((T_CHUNK * B, 1), lambda g: (g, 0)),
            pl.BlockSpec((HID, HID), lambda g: (0, 0)),
            pl.BlockSpec((1, HID), lambda g: (0, 0)),
        ],
        out_specs=[
            pl.BlockSpec((T_CHUNK * B, HID), lambda g: (g, 0)),
            pl.BlockSpec((B, HID), lambda g: (0, 0)),
        ],
        out_shape=[
            jax.ShapeDtypeStruct((S * B, HID), jnp.float32),
            jax.ShapeDtypeStruct((B, HID), jnp.float32),
        ],
        scratch_shapes=[pltpu.VMEM((B, HID), jnp.float32)],
    )(a, mask, w_hh, b.reshape(1, HID))


def kernel(x, embedding, W_ih0, W_hh0, b0, W_ih1, W_hh1, b1):
    xt = x.T
    idx = xt.reshape(-1).astype(jnp.int32)
    mask = (xt != 0).astype(jnp.float32).reshape(S * B, 1)

    xe = _sc_gather(embedding, idx)
    a0 = _matmul(xe, W_ih0)
    h0_all, _ = _recurrence(a0, mask, W_hh0, b0)
    a1 = _matmul(h0_all, W_ih1)
    _, h1 = _recurrence(a1, mask, W_hh1, b1)
    return h1
